# baseline (speedup 1.0000x reference)
import functools

import jax
import jax.numpy as jnp
from jax import lax
from jax.experimental import pallas as pl
from jax.experimental.pallas import tpu as pltpu
from jax.experimental.pallas import tpu_sc as plsc

B = 4096
F = 26
D = 16
DFM = 5
HID = 20
FP = 32
NC, NS = 2, 16
NW = NC * NS
BPW = B // NW
RPW = FP * BPW
NSTR = RPW // 128


def _sc_gather(table, idx_r):
  mesh = plsc.VectorSubcoreMesh(core_axis_name="c", subcore_axis_name="s",
                                num_cores=NC, num_subcores=NS)

  @functools.partial(
      pl.kernel,
      out_type=jax.ShapeDtypeStruct((B * FP, D), jnp.float32),
      mesh=mesh,
      compiler_params=pltpu.CompilerParams(use_tc_tiling_on_sc=False),
      scratch_types=[
          pltpu.VMEM((NSTR, 128), jnp.int32),
          pltpu.VMEM((RPW, D), jnp.float32),
          pltpu.SemaphoreType.DMA,
      ],
  )
  def gather(idx_hbm, table_hbm, out_hbm, idx_v, buf, sem):
    wid = lax.axis_index("s") * NC + lax.axis_index("c")
    pltpu.sync_copy(idx_hbm.at[wid], idx_v)
    def grp(t, carry):
      cps = [pltpu.async_copy(table_hbm.at[idx_v.at[t * 8 + k]],
                              buf.at[pl.ds((t * 8 + k) * 128, 128)], sem)
             for k in range(8)]
      for c in cps:
        c.wait()
      return carry
    lax.fori_loop(0, NSTR // 8, grp, 0)
    pltpu.sync_copy(buf, out_hbm.at[pl.ds(wid * RPW, RPW)])

  return gather(idx_r, table)


def _tc_dense(emb512, dense_features, labels2, G512, M512, Wd0, W_dense,
              b_dense, b_deep, W1a, w1row, b1, W2, b2):
  def body(emb_ref, dense_ref, lab_ref, g_ref, m_ref, wd0_ref, wdn_ref,
           bdn_ref, bdp_ref, w1_ref, w1r_ref, b1_ref, w2_ref, b2_ref,
           loss_ref, p_ref):
    e = emb_ref[...]
    acc = jnp.dot(e, g_ref[...], preferred_element_type=jnp.float32)
    sq = jnp.dot(e * e, m_ref[...], preferred_element_type=jnp.float32)
    dense_e = jnp.maximum(
        jnp.dot(dense_ref[...], wdn_ref[...],
                preferred_element_type=jnp.float32) + bdn_ref[...], 0.0)
    s = acc[:, 0:D] + dense_e
    deep = jnp.maximum(
        acc[:, D:D + DFM]
        + jnp.dot(dense_e, wd0_ref[...], preferred_element_type=jnp.float32)
        + bdp_ref[...], 0.0)
    fmv = s * s - (sq + dense_e * dense_e)
    fm = 0.5 * jnp.dot(fmv, jnp.ones((D, 1), jnp.float32),
                       preferred_element_type=jnp.float32)
    h = jnp.maximum(
        jnp.dot(deep, w1_ref[...], preferred_element_type=jnp.float32)
        + fm * w1r_ref[...] + b1_ref[...], 0.0)
    logits = jnp.dot(h, w2_ref[...],
                     preferred_element_type=jnp.float32) + b2_ref[...]
    p = 1.0 / (1.0 + jnp.exp(-logits))
    p = jnp.clip(p, 1e-7, 1.0 - 1e-7)
    lab = lab_ref[...]
    ll = lab * jnp.log(p) + (1.0 - lab) * jnp.log(1.0 - p)
    loss_ref[...] = jnp.broadcast_to(-jnp.sum(ll) * (1.0 / B), (1, 1))
    p_ref[...] = p

  return pl.pallas_call(
      body,
      out_shape=(jax.ShapeDtypeStruct((1, 1), jnp.float32),
                 jax.ShapeDtypeStruct((B, 1), jnp.float32)),
  )(emb512, dense_features, labels2, G512, M512, Wd0, W_dense,
    b_dense, b_deep, W1a, w1row, b1, W2, b2)


def kernel(dense_features, sparse_features, permu, labels, table, W_dense,
           b_dense, W_deep, b_deep, W_over1, b_over1, W_over2, b_over2):
  idx = jnp.take(sparse_features, permu, axis=1).astype(jnp.int32)
  idx = jnp.pad(idx, ((0, 0), (0, FP - F)))
  idx_r = idx.reshape(NW, NSTR, 128)

  rows = _sc_gather(table, idx_r)
  emb512 = rows.reshape(B, FP * D)

  Wd_emb = W_deep[D:(F + 1) * D].reshape(F, D, DFM)
  Wd512 = jnp.pad(Wd_emb, ((0, FP - F), (0, 0), (0, 0))).reshape(FP * D, DFM)
  eye = jnp.broadcast_to(jnp.eye(D, dtype=jnp.float32)[None], (FP, D, D))
  msk = (jnp.arange(FP) < F).astype(jnp.float32)[:, None, None]
  M512 = (eye * msk).reshape(FP * D, D)
  G512 = jnp.concatenate([M512, Wd512], axis=1)

  loss, p = _tc_dense(
      emb512, dense_features, labels.reshape(B, 1), G512, M512,
      W_deep[0:D], W_dense, b_dense.reshape(1, D), b_deep.reshape(1, DFM),
      W_over1[0:DFM], W_over1[DFM:DFM + 1], b_over1.reshape(1, HID),
      W_over2, b_over2.reshape(1, 1))
  return (loss.reshape(()), p.reshape(B), labels)
